# Initial kernel scaffold; baseline (speedup 1.0000x reference)
import functools

import jax
import jax.numpy as jnp
from jax import lax
from jax.experimental import pallas as pl
from jax.experimental.pallas import tpu as pltpu
from jax.experimental.pallas import tpu_sc as plsc

_NUM_ENTITIES = 1000000
_D = 64
_N_NODES = 81920
_NQ = 4096
_NNEG = 50
_NW = 32
_W = 128

_NODE_CH = _N_NODES // _NW // _W
_CORR_CH = _NQ * _NNEG // _NW // _W
_QPW = _NQ // _NW
_RPT = _NQ // 16


def _sc_gather_accumulate(x2d, batch2d, corr2d, qid2d, answers, table):
    mesh = plsc.VectorSubcoreMesh(core_axis_name="c", subcore_axis_name="s")
    out_type = [
        jax.ShapeDtypeStruct((2, _NQ, _D), jnp.float32),
        jax.ShapeDtypeStruct((2, _NQ, _D), jnp.float32),
        jax.ShapeDtypeStruct((2, _NQ, 16), jnp.float32),
        jax.ShapeDtypeStruct((_NQ, _D), jnp.float32),
    ]
    scratch = [
        pltpu.VMEM((_NODE_CH, _W), jnp.int32),
        pltpu.VMEM((_NODE_CH, _W), jnp.int32),
        pltpu.VMEM((_CORR_CH, _W), jnp.int32),
        pltpu.VMEM((_CORR_CH, _W), jnp.int32),
        pltpu.VMEM((_QPW,), jnp.int32),
        pltpu.VMEM((_W, _D), jnp.float32),
        pltpu.VMEM((_W, 16), jnp.float32),
        pltpu.VMEM((_RPT, _D), jnp.float32),
        pltpu.VMEM((_RPT, 16), jnp.float32),
        pltpu.VMEM_SHARED((_NQ, _D), jnp.float32),
        pltpu.VMEM_SHARED((_NQ, _D), jnp.float32),
        pltpu.VMEM_SHARED((_NQ, 16), jnp.float32),
        pltpu.SemaphoreType.DMA,
    ]

    @functools.partial(pl.kernel, out_type=out_type, mesh=mesh,
                       scratch_types=scratch)
    def k(x_h, b_h, c_h, q_h, a_h, t_h, qsum_h, csum_h, cnt_h, ae_h,
          xi, bi, ci, qi, ai, rows, ones, z64, z16, qacc, cacc, ctacc, sem):
        cid = lax.axis_index("c")
        sid = lax.axis_index("s")
        w = cid * 16 + sid

        zero16 = jnp.zeros((16,), jnp.float32)
        one16 = jnp.full((16,), 1.0, jnp.float32)

        @pl.loop(0, _RPT)
        def _(r):
            for k4 in range(_D // 16):
                z64[r, pl.ds(k4 * 16, 16)] = zero16
            z16[r, :] = zero16

        @pl.loop(0, _W)
        def _(r):
            ones[r, :] = one16

        row0 = sid * _RPT
        pltpu.sync_copy(z64, qacc.at[pl.ds(row0, _RPT)])
        pltpu.sync_copy(z64, cacc.at[pl.ds(row0, _RPT)])
        pltpu.sync_copy(z16, ctacc.at[pl.ds(row0, _RPT)])
        plsc.subcore_barrier()

        pltpu.sync_copy(x_h.at[pl.ds(w * _NODE_CH, _NODE_CH)], xi)
        pltpu.sync_copy(b_h.at[pl.ds(w * _NODE_CH, _NODE_CH)], bi)
        pltpu.sync_copy(c_h.at[pl.ds(w * _CORR_CH, _CORR_CH)], ci)
        pltpu.sync_copy(q_h.at[pl.ds(w * _CORR_CH, _CORR_CH)], qi)
        pltpu.sync_copy(a_h.at[pl.ds(w * _QPW, _QPW)], ai)

        @pl.loop(0, _NODE_CH)
        def _(j):
            pltpu.async_copy(t_h.at[xi.at[j]], rows, sem).wait()
            pltpu.sync_copy(rows, qacc.at[bi.at[j]], add=True)
            pltpu.sync_copy(ones, ctacc.at[bi.at[j]], add=True)

        @pl.loop(0, _CORR_CH)
        def _(j):
            pltpu.async_copy(t_h.at[ci.at[j]], rows, sem).wait()
            pltpu.sync_copy(rows, cacc.at[qi.at[j]], add=True)

        pltpu.async_copy(t_h.at[ai], rows, sem).wait()
        pltpu.sync_copy(rows, ae_h.at[pl.ds(w * _QPW, _QPW)])

        plsc.subcore_barrier()
        pltpu.sync_copy(qacc.at[pl.ds(row0, _RPT)],
                        qsum_h.at[cid, pl.ds(row0, _RPT)])
        pltpu.sync_copy(cacc.at[pl.ds(row0, _RPT)],
                        csum_h.at[cid, pl.ds(row0, _RPT)])
        pltpu.sync_copy(ctacc.at[pl.ds(row0, _RPT)],
                        cnt_h.at[cid, pl.ds(row0, _RPT)])

    return k(x2d, batch2d, corr2d, qid2d, answers, table)


def _tc_finish(qsum, csum, cnt, ansemb):
    def body(qs_ref, cs_ref, cn_ref, ae_ref, loss_ref, gold_ref, corr_ref):
        qs = qs_ref[0] + qs_ref[1]
        cs = cs_ref[0] + cs_ref[1]
        cn = cn_ref[0] + cn_ref[1]
        count = jnp.sum(cn, axis=1) * (1.0 / 16.0)
        query = qs / jnp.maximum(count, 1.0)[:, None]
        gold = jnp.sum(query * ae_ref[...], axis=1)
        corr = jnp.sum(query * cs, axis=1) * (1.0 / _NNEG)
        loss_ref[...] = jnp.mean(
            jnp.maximum(1.0 + corr - gold, 0.0)).reshape(1, 1)
        gold_ref[...] = gold
        corr_ref[...] = corr

    return pl.pallas_call(
        body,
        out_shape=[
            jax.ShapeDtypeStruct((1, 1), jnp.float32),
            jax.ShapeDtypeStruct((_NQ,), jnp.float32),
            jax.ShapeDtypeStruct((_NQ,), jnp.float32),
        ],
    )(qsum, csum, cnt, ansemb)


def kernel(x, batch, answers, corrupted, table):
    x2d = x.astype(jnp.int32).reshape(_N_NODES // _W, _W)
    batch2d = batch.astype(jnp.int32).reshape(_N_NODES // _W, _W)
    corr2d = corrupted.astype(jnp.int32).reshape(_NQ * _NNEG // _W, _W)
    qid2d = (jnp.arange(_NQ * _NNEG, dtype=jnp.int32) // _NNEG).reshape(
        _NQ * _NNEG // _W, _W)
    ans = answers.astype(jnp.int32)
    tab = table.astype(jnp.float32)

    qsum, csum, cnt, ae = _sc_gather_accumulate(
        x2d, batch2d, corr2d, qid2d, ans, tab)
    loss, gold, corr = _tc_finish(qsum, csum, cnt, ae)
    return (loss.reshape(()), gold, corr)


# speedup vs baseline: 1.0700x; 1.0700x over previous
import functools

import jax
import jax.numpy as jnp
from jax import lax
from jax.experimental import pallas as pl
from jax.experimental.pallas import tpu as pltpu
from jax.experimental.pallas import tpu_sc as plsc

_NUM_ENTITIES = 1000000
_D = 64
_N_NODES = 81920
_NQ = 4096
_NNEG = 50
_NW = 32
_W = 128

_NODE_CH = _N_NODES // _NW // _W
_CORR_CH = _NQ * _NNEG // _NW // _W
_QPW = _NQ // _NW
_RPT = _NQ // 16


def _sc_gather_accumulate(x2d, batch2d, corr2d, qid2d, answers, table):
    mesh = plsc.VectorSubcoreMesh(core_axis_name="c", subcore_axis_name="s")
    out_type = [
        jax.ShapeDtypeStruct((2, _NQ, _D), jnp.float32),
        jax.ShapeDtypeStruct((2, _NQ, _D), jnp.float32),
        jax.ShapeDtypeStruct((2, _NQ, 16), jnp.float32),
        jax.ShapeDtypeStruct((_NQ, _D), jnp.float32),
    ]
    scratch = [
        pltpu.VMEM((_NODE_CH, _W), jnp.int32),
        pltpu.VMEM((_NODE_CH, _W), jnp.int32),
        pltpu.VMEM((_CORR_CH, _W), jnp.int32),
        pltpu.VMEM((_CORR_CH, _W), jnp.int32),
        pltpu.VMEM((_QPW,), jnp.int32),
        pltpu.VMEM((_W, _D), jnp.float32),
        pltpu.VMEM((_W, 16), jnp.float32),
        pltpu.VMEM((_RPT, _D), jnp.float32),
        pltpu.VMEM((_RPT, 16), jnp.float32),
        pltpu.VMEM_SHARED((_NQ, _D), jnp.float32),
        pltpu.VMEM_SHARED((_NQ, _D), jnp.float32),
        pltpu.VMEM_SHARED((_NQ, 16), jnp.float32),
        pltpu.SemaphoreType.DMA,
    ]

    @functools.partial(pl.kernel, out_type=out_type, mesh=mesh,
                       scratch_types=scratch,
                       compiler_params=pltpu.CompilerParams(
                           use_tc_tiling_on_sc=False))
    def k(x_h, b_h, c_h, q_h, a_h, t_h, qsum_h, csum_h, cnt_h, ae_h,
          xi, bi, ci, qi, ai, rows, ones, z64, z16, qacc, cacc, ctacc, sem):
        cid = lax.axis_index("c")
        sid = lax.axis_index("s")
        w = cid * 16 + sid

        zero16 = jnp.zeros((16,), jnp.float32)
        one16 = jnp.full((16,), 1.0, jnp.float32)

        @pl.loop(0, _RPT)
        def _(r):
            for k4 in range(_D // 16):
                z64[r, pl.ds(k4 * 16, 16)] = zero16
            z16[r, :] = zero16

        @pl.loop(0, _W)
        def _(r):
            ones[r, :] = one16

        row0 = sid * _RPT
        pltpu.sync_copy(z64, qacc.at[pl.ds(row0, _RPT)])
        pltpu.sync_copy(z64, cacc.at[pl.ds(row0, _RPT)])
        pltpu.sync_copy(z16, ctacc.at[pl.ds(row0, _RPT)])
        plsc.subcore_barrier()

        pltpu.sync_copy(x_h.at[w], xi)
        pltpu.sync_copy(b_h.at[w], bi)
        pltpu.sync_copy(c_h.at[w], ci)
        pltpu.sync_copy(q_h.at[w], qi)
        pltpu.sync_copy(a_h.at[pl.ds(w * _QPW, _QPW)], ai)

        @pl.loop(0, _NODE_CH)
        def _(j):
            pltpu.async_copy(t_h.at[xi.at[j]], rows, sem).wait()
            pltpu.sync_copy(rows, qacc.at[bi.at[j]], add=True)
            pltpu.sync_copy(ones, ctacc.at[bi.at[j]], add=True)

        @pl.loop(0, _CORR_CH)
        def _(j):
            pltpu.async_copy(t_h.at[ci.at[j]], rows, sem).wait()
            pltpu.sync_copy(rows, cacc.at[qi.at[j]], add=True)

        pltpu.async_copy(t_h.at[ai], rows, sem).wait()
        pltpu.sync_copy(rows, ae_h.at[pl.ds(w * _QPW, _QPW)])

        plsc.subcore_barrier()
        pltpu.sync_copy(qacc.at[pl.ds(row0, _RPT)],
                        qsum_h.at[cid, pl.ds(row0, _RPT)])
        pltpu.sync_copy(cacc.at[pl.ds(row0, _RPT)],
                        csum_h.at[cid, pl.ds(row0, _RPT)])
        pltpu.sync_copy(ctacc.at[pl.ds(row0, _RPT)],
                        cnt_h.at[cid, pl.ds(row0, _RPT)])

    return k(x2d, batch2d, corr2d, qid2d, answers, table)


def _tc_finish(qsum, csum, cnt, ansemb):
    def body(qs_ref, cs_ref, cn_ref, ae_ref, loss_ref, gold_ref, corr_ref):
        qs = qs_ref[0] + qs_ref[1]
        cs = cs_ref[0] + cs_ref[1]
        cn = cn_ref[0] + cn_ref[1]
        count = jnp.sum(cn, axis=1) * (1.0 / 16.0)
        query = qs / jnp.maximum(count, 1.0)[:, None]
        gold = jnp.sum(query * ae_ref[...], axis=1)
        corr = jnp.sum(query * cs, axis=1) * (1.0 / _NNEG)
        loss_ref[...] = jnp.mean(
            jnp.maximum(1.0 + corr - gold, 0.0)).reshape(1, 1)
        gold_ref[...] = gold
        corr_ref[...] = corr

    return pl.pallas_call(
        body,
        out_shape=[
            jax.ShapeDtypeStruct((1, 1), jnp.float32),
            jax.ShapeDtypeStruct((_NQ,), jnp.float32),
            jax.ShapeDtypeStruct((_NQ,), jnp.float32),
        ],
    )(qsum, csum, cnt, ansemb)


def kernel(x, batch, answers, corrupted, table):
    x2d = x.astype(jnp.int32).reshape(_NW, _NODE_CH, _W)
    batch2d = batch.astype(jnp.int32).reshape(_NW, _NODE_CH, _W)
    corr2d = corrupted.astype(jnp.int32).reshape(_NW, _CORR_CH, _W)
    qid2d = (jnp.arange(_NQ * _NNEG, dtype=jnp.int32) // _NNEG).reshape(
        _NW, _CORR_CH, _W)
    ans = answers.astype(jnp.int32)
    tab = table.astype(jnp.float32)

    qsum, csum, cnt, ae = _sc_gather_accumulate(
        x2d, batch2d, corr2d, qid2d, ans, tab)
    loss, gold, corr = _tc_finish(qsum, csum, cnt, ae)
    return (loss.reshape(()), gold, corr)


# speedup vs baseline: 1.6558x; 1.5475x over previous
import functools

import jax
import jax.numpy as jnp
from jax import lax
from jax.experimental import pallas as pl
from jax.experimental.pallas import tpu as pltpu
from jax.experimental.pallas import tpu_sc as plsc

_D = 64
_N_NODES = 81920
_NQ = 4096
_NNEG = 50
_NW = 32
_W = 128

_L = 4096
_TGRID = 123
_H = _TGRID * _L

_NODE_CH = _N_NODES // _NW // _W
_CORR_CH = _NQ * _NNEG // _NW // _W
_QPW = _NQ // _NW
_ACC_R = 2 * _NQ
_RPT = _ACC_R // 16


def _tc_repack(tabT):
    def body(a_ref, b_ref, out_ref):
        out_ref[:, 0:_D] = a_ref[...].T
        out_ref[:, _D:128] = b_ref[...].T

    return pl.pallas_call(
        body,
        grid=(_TGRID,),
        in_specs=[pl.BlockSpec((_D, _L), lambda i: (0, i)),
                  pl.BlockSpec((_D, _L),
                               lambda i: (0, jnp.minimum(i + _TGRID, 244)))],
        out_specs=pl.BlockSpec((_L, 128), lambda i: (i, 0)),
        out_shape=jax.ShapeDtypeStruct((_H, 128), jnp.float32),
    )(tabT, tabT)


def _sc_gather_accumulate(gx, sx, gc, sc_idx, ga, packed):
    mesh = plsc.VectorSubcoreMesh(core_axis_name="c", subcore_axis_name="s")
    out_type = [
        jax.ShapeDtypeStruct((2, _ACC_R, 128), jnp.float32),
        jax.ShapeDtypeStruct((2, _ACC_R, 128), jnp.float32),
        jax.ShapeDtypeStruct((2, _ACC_R, 16), jnp.float32),
        jax.ShapeDtypeStruct((_NQ, 128), jnp.float32),
    ]
    scratch = [
        pltpu.VMEM((_NODE_CH, _W), jnp.int32),
        pltpu.VMEM((_NODE_CH, _W), jnp.int32),
        pltpu.VMEM((_CORR_CH, _W), jnp.int32),
        pltpu.VMEM((_CORR_CH, _W), jnp.int32),
        pltpu.VMEM((_QPW,), jnp.int32),
        pltpu.VMEM((_W, 128), jnp.float32),
        pltpu.VMEM((_W, 16), jnp.float32),
        pltpu.VMEM((_W, 128), jnp.float32),
        pltpu.VMEM((_W, 16), jnp.float32),
        pltpu.VMEM_SHARED((_ACC_R, 128), jnp.float32),
        pltpu.VMEM_SHARED((_ACC_R, 16), jnp.float32),
        pltpu.SemaphoreType.DMA,
    ]

    @functools.partial(pl.kernel, out_type=out_type, mesh=mesh,
                       scratch_types=scratch,
                       compiler_params=pltpu.CompilerParams(
                           use_tc_tiling_on_sc=False))
    def k(gx_h, sx_h, gc_h, sc_h, ga_h, t_h, qsum_h, csum_h, cnt_h, aw_h,
          gxi, sxi, gci, sci, gai, rows, ones, z128, z16, acc, cnt, sem):
        cid = lax.axis_index("c")
        sid = lax.axis_index("s")
        w = cid * 16 + sid
        row0 = sid * _RPT

        zero16 = jnp.zeros((16,), jnp.float32)
        one16 = jnp.full((16,), 1.0, jnp.float32)

        @pl.loop(0, _W)
        def _(r):
            for k8 in range(128 // 16):
                z128[r, pl.ds(k8 * 16, 16)] = zero16
            z16[r, :] = zero16
            ones[r, :] = one16

        for k4 in range(_RPT // _W):
            pltpu.sync_copy(z128, acc.at[pl.ds(row0 + k4 * _W, _W)])
            pltpu.sync_copy(z16, cnt.at[pl.ds(row0 + k4 * _W, _W)])

        pltpu.sync_copy(gx_h.at[w], gxi)
        pltpu.sync_copy(sx_h.at[w], sxi)
        pltpu.sync_copy(gc_h.at[w], gci)
        pltpu.sync_copy(sc_h.at[w], sci)
        pltpu.sync_copy(ga_h.at[pl.ds(w * _QPW, _QPW)], gai)
        plsc.subcore_barrier()

        @pl.loop(0, _NODE_CH)
        def _(j):
            pltpu.async_copy(t_h.at[gxi.at[j]], rows, sem).wait()
            pltpu.sync_copy(rows, acc.at[sxi.at[j]], add=True)
            pltpu.sync_copy(ones, cnt.at[sxi.at[j]], add=True)

        plsc.subcore_barrier()
        pltpu.sync_copy(acc.at[pl.ds(row0, _RPT)],
                        qsum_h.at[cid, pl.ds(row0, _RPT)])
        pltpu.sync_copy(cnt.at[pl.ds(row0, _RPT)],
                        cnt_h.at[cid, pl.ds(row0, _RPT)])
        for k4 in range(_RPT // _W):
            pltpu.sync_copy(z128, acc.at[pl.ds(row0 + k4 * _W, _W)])
        plsc.subcore_barrier()

        @pl.loop(0, _CORR_CH)
        def _(j):
            pltpu.async_copy(t_h.at[gci.at[j]], rows, sem).wait()
            pltpu.sync_copy(rows, acc.at[sci.at[j]], add=True)

        pltpu.async_copy(t_h.at[gai], rows, sem).wait()
        pltpu.sync_copy(rows, aw_h.at[pl.ds(w * _QPW, _QPW)])

        plsc.subcore_barrier()
        pltpu.sync_copy(acc.at[pl.ds(row0, _RPT)],
                        csum_h.at[cid, pl.ds(row0, _RPT)])

    return k(gx, sx, gc, sc_idx, ga, packed)


def _tc_finish(qsum, csum, cnt, aw, answers):
    def body(q_ref, c_ref, n_ref, a_ref, ai_ref, loss_ref, gold_ref,
             corr_ref):
        def halves(r):
            return (r[0, :_NQ, :_D] + r[0, _NQ:, _D:]
                    + r[1, :_NQ, :_D] + r[1, _NQ:, _D:])

        qs = halves(q_ref)
        cs = halves(c_ref)
        n = n_ref[...]
        count = jnp.sum(n[0, :_NQ] + n[0, _NQ:] + n[1, :_NQ] + n[1, _NQ:],
                        axis=1) * (1.0 / 16.0)
        query = qs / jnp.maximum(count, 1.0)[:, None]
        hi = ai_ref[...] >= _H
        ae = jnp.where(hi, a_ref[:, _D:], a_ref[:, :_D])
        gold = jnp.sum(query * ae, axis=1)
        corr = jnp.sum(query * cs, axis=1) * (1.0 / _NNEG)
        loss_ref[...] = jnp.mean(
            jnp.maximum(1.0 + corr - gold, 0.0)).reshape(1, 1)
        gold_ref[...] = gold
        corr_ref[...] = corr

    return pl.pallas_call(
        body,
        out_shape=[
            jax.ShapeDtypeStruct((1, 1), jnp.float32),
            jax.ShapeDtypeStruct((_NQ,), jnp.float32),
            jax.ShapeDtypeStruct((_NQ,), jnp.float32),
        ],
    )(qsum, csum, cnt, aw, answers.reshape(_NQ, 1))


def kernel(x, batch, answers, corrupted, table):
    xf = x.astype(jnp.int32).reshape(_N_NODES)
    bf = batch.astype(jnp.int32)
    cf = corrupted.astype(jnp.int32).reshape(_NQ * _NNEG)
    af = answers.astype(jnp.int32)
    qid = jnp.arange(_NQ * _NNEG, dtype=jnp.int32) // _NNEG

    gx = jnp.where(xf >= _H, xf - _H, xf).reshape(_NW, _NODE_CH, _W)
    sx = (bf + _NQ * (xf >= _H)).reshape(_NW, _NODE_CH, _W)
    gc = jnp.where(cf >= _H, cf - _H, cf).reshape(_NW, _CORR_CH, _W)
    sc_idx = (qid + _NQ * (cf >= _H)).reshape(_NW, _CORR_CH, _W)
    ga = jnp.where(af >= _H, af - _H, af)

    packed = _tc_repack(table.astype(jnp.float32).T)
    qsum, csum, cnt, aw = _sc_gather_accumulate(gx, sx, gc, sc_idx, ga,
                                                packed)
    loss, gold, corr = _tc_finish(qsum, csum, cnt, aw, af)
    return (loss.reshape(()), gold, corr)


# speedup vs baseline: 1.9286x; 1.1648x over previous
import functools

import jax
import jax.numpy as jnp
from jax import lax
from jax.experimental import pallas as pl
from jax.experimental.pallas import tpu as pltpu
from jax.experimental.pallas import tpu_sc as plsc

_D = 64
_N_NODES = 81920
_NQ = 4096
_NNEG = 50
_NW = 32
_W = 128

_L = 4096
_TGRID = 123
_H = _TGRID * _L
_LAST_B = 1000001 // _L

_NODE_CH = _N_NODES // _NW // _W
_CORR_CH = _NQ * _NNEG // _NW // _W
_QPW = _NQ // _NW
_RPT = _NQ // 16


def _tc_repack(tabT):
    def body(a_ref, b_ref, out_ref):
        out_ref[:, 0:_D] = a_ref[...].T
        out_ref[:, _D:128] = b_ref[...].T

    return pl.pallas_call(
        body,
        grid=(_TGRID,),
        in_specs=[pl.BlockSpec((_D, _L), lambda i: (0, i)),
                  pl.BlockSpec((_D, _L),
                               lambda i: (0, jnp.minimum(i + _TGRID,
                                                         _LAST_B)))],
        out_specs=pl.BlockSpec((_L, 128), lambda i: (i, 0)),
        out_shape=jax.ShapeDtypeStruct((_H, 128), jnp.float32),
    )(tabT, tabT)


def _sc_gather_accumulate(gx, bx, gc, qc, ga, packed64):
    mesh = plsc.VectorSubcoreMesh(core_axis_name="c", subcore_axis_name="s")
    out_type = [
        jax.ShapeDtypeStruct((2, _NQ, _D), jnp.float32),
        jax.ShapeDtypeStruct((2, _NQ, _D), jnp.float32),
        jax.ShapeDtypeStruct((2, _NQ, 16), jnp.float32),
        jax.ShapeDtypeStruct((_NQ, _D), jnp.float32),
    ]
    scratch = [
        pltpu.VMEM((_NODE_CH, _W), jnp.int32),
        pltpu.VMEM((_NODE_CH, _W), jnp.int32),
        pltpu.VMEM((_CORR_CH, _W), jnp.int32),
        pltpu.VMEM((_CORR_CH, _W), jnp.int32),
        pltpu.VMEM((_QPW,), jnp.int32),
        pltpu.VMEM((_W, _D), jnp.float32),
        pltpu.VMEM((_W, 16), jnp.float32),
        pltpu.VMEM((_RPT, _D), jnp.float32),
        pltpu.VMEM((_RPT, 16), jnp.float32),
        pltpu.VMEM_SHARED((_NQ, _D), jnp.float32),
        pltpu.VMEM_SHARED((_NQ, _D), jnp.float32),
        pltpu.VMEM_SHARED((_NQ, 16), jnp.float32),
        pltpu.SemaphoreType.DMA,
    ]

    @functools.partial(pl.kernel, out_type=out_type, mesh=mesh,
                       scratch_types=scratch,
                       compiler_params=pltpu.CompilerParams(
                           use_tc_tiling_on_sc=False))
    def k(gx_h, bx_h, gc_h, qc_h, ga_h, t_h, qsum_h, csum_h, cnt_h, ae_h,
          gxi, bxi, gci, qci, gai, rows, ones, z64, z16, qacc, cacc, ctacc,
          sem):
        cid = lax.axis_index("c")
        sid = lax.axis_index("s")
        w = cid * 16 + sid
        row0 = sid * _RPT

        zero16 = jnp.zeros((16,), jnp.float32)
        one16 = jnp.full((16,), 1.0, jnp.float32)

        @pl.loop(0, _RPT)
        def _(r):
            for k4 in range(_D // 16):
                z64[r, pl.ds(k4 * 16, 16)] = zero16
            z16[r, :] = zero16

        @pl.loop(0, _W)
        def _(r):
            ones[r, :] = one16

        pltpu.sync_copy(z64, qacc.at[pl.ds(row0, _RPT)])
        pltpu.sync_copy(z64, cacc.at[pl.ds(row0, _RPT)])
        pltpu.sync_copy(z16, ctacc.at[pl.ds(row0, _RPT)])

        pltpu.sync_copy(gx_h.at[w], gxi)
        pltpu.sync_copy(bx_h.at[w], bxi)
        pltpu.sync_copy(gc_h.at[w], gci)
        pltpu.sync_copy(qc_h.at[w], qci)
        pltpu.sync_copy(ga_h.at[pl.ds(w * _QPW, _QPW)], gai)
        plsc.subcore_barrier()

        @pl.loop(0, _NODE_CH)
        def _(j):
            pltpu.async_copy(t_h.at[gxi.at[j]], rows, sem).wait()
            pltpu.sync_copy(rows, qacc.at[bxi.at[j]], add=True)
            pltpu.sync_copy(ones, ctacc.at[bxi.at[j]], add=True)

        @pl.loop(0, _CORR_CH)
        def _(j):
            pltpu.async_copy(t_h.at[gci.at[j]], rows, sem).wait()
            pltpu.sync_copy(rows, cacc.at[qci.at[j]], add=True)

        pltpu.async_copy(t_h.at[gai], rows, sem).wait()
        pltpu.sync_copy(rows, ae_h.at[pl.ds(w * _QPW, _QPW)])

        plsc.subcore_barrier()
        pltpu.sync_copy(qacc.at[pl.ds(row0, _RPT)],
                        qsum_h.at[cid, pl.ds(row0, _RPT)])
        pltpu.sync_copy(cacc.at[pl.ds(row0, _RPT)],
                        csum_h.at[cid, pl.ds(row0, _RPT)])
        pltpu.sync_copy(ctacc.at[pl.ds(row0, _RPT)],
                        cnt_h.at[cid, pl.ds(row0, _RPT)])

    return k(gx, bx, gc, qc, ga, packed64)


def _tc_finish(qsum, csum, cnt, ansemb):
    def body(qs_ref, cs_ref, cn_ref, ae_ref, loss_ref, gold_ref, corr_ref):
        qs = qs_ref[0] + qs_ref[1]
        cs = cs_ref[0] + cs_ref[1]
        cn = cn_ref[0] + cn_ref[1]
        count = jnp.sum(cn, axis=1) * (1.0 / 16.0)
        query = qs / jnp.maximum(count, 1.0)[:, None]
        gold = jnp.sum(query * ae_ref[...], axis=1)
        corr = jnp.sum(query * cs, axis=1) * (1.0 / _NNEG)
        loss_ref[...] = jnp.mean(
            jnp.maximum(1.0 + corr - gold, 0.0)).reshape(1, 1)
        gold_ref[...] = gold
        corr_ref[...] = corr

    return pl.pallas_call(
        body,
        out_shape=[
            jax.ShapeDtypeStruct((1, 1), jnp.float32),
            jax.ShapeDtypeStruct((_NQ,), jnp.float32),
            jax.ShapeDtypeStruct((_NQ,), jnp.float32),
        ],
    )(qsum, csum, cnt, ansemb)


def _packed_row(e):
    return jnp.where(e < _H, 2 * e, 2 * (e - _H) + 1)


def kernel(x, batch, answers, corrupted, table):
    xf = x.astype(jnp.int32).reshape(_N_NODES)
    bf = batch.astype(jnp.int32)
    cf = corrupted.astype(jnp.int32).reshape(_NQ * _NNEG)
    af = answers.astype(jnp.int32)
    qid = jnp.arange(_NQ * _NNEG, dtype=jnp.int32) // _NNEG

    gx = _packed_row(xf).reshape(_NW, _NODE_CH, _W)
    bx = bf.reshape(_NW, _NODE_CH, _W)
    gc = _packed_row(cf).reshape(_NW, _CORR_CH, _W)
    qc = qid.reshape(_NW, _CORR_CH, _W)
    ga = _packed_row(af)

    packed = _tc_repack(table.astype(jnp.float32).T)
    packed64 = packed.reshape(2 * _H, _D)
    qsum, csum, cnt, ae = _sc_gather_accumulate(gx, bx, gc, qc, ga, packed64)
    loss, gold, corr = _tc_finish(qsum, csum, cnt, ae)
    return (loss.reshape(()), gold, corr)


# speedup vs baseline: 2.1867x; 1.1338x over previous
import functools

import jax
import jax.numpy as jnp
from jax import lax
from jax.experimental import pallas as pl
from jax.experimental.pallas import tpu as pltpu
from jax.experimental.pallas import tpu_sc as plsc

_D = 64
_N_NODES = 81920
_NQ = 4096
_NNEG = 50
_NW = 32
_W = 128

_L = 4096
_TGRID = 123
_H = _TGRID * _L
_LAST_B = 1000001 // _L

_NODE_CH = _N_NODES // _NW // _W
_CORR_CH = _NQ * _NNEG // _NW // _W
_NB = 5
_QPW = _NQ // _NW
_RPT = _NQ // 16


def _tc_repack(tabT):
    def body(a_ref, b_ref, out_ref):
        out_ref[:, 0:_D] = a_ref[...].T
        out_ref[:, _D:128] = b_ref[...].T

    return pl.pallas_call(
        body,
        grid=(_TGRID,),
        in_specs=[pl.BlockSpec((_D, _L), lambda i: (0, i)),
                  pl.BlockSpec((_D, _L),
                               lambda i: (0, jnp.minimum(i + _TGRID,
                                                         _LAST_B)))],
        out_specs=pl.BlockSpec((_L, 128), lambda i: (i, 0)),
        out_shape=jax.ShapeDtypeStruct((_H, 128), jnp.float32),
    )(tabT, tabT)


def _sc_gather_accumulate(gx, bx, gc, qc, ga, packed64):
    mesh = plsc.VectorSubcoreMesh(core_axis_name="c", subcore_axis_name="s")
    out_type = [
        jax.ShapeDtypeStruct((2, _NQ, _D), jnp.float32),
        jax.ShapeDtypeStruct((2, _NQ, _D), jnp.float32),
        jax.ShapeDtypeStruct((2, _NQ, 16), jnp.float32),
        jax.ShapeDtypeStruct((_NQ, _D), jnp.float32),
    ]
    scratch = [
        pltpu.VMEM((_NODE_CH, _W), jnp.int32),
        pltpu.VMEM((_NODE_CH, _W), jnp.int32),
        pltpu.VMEM((_CORR_CH, _W), jnp.int32),
        pltpu.VMEM((_CORR_CH, _W), jnp.int32),
        pltpu.VMEM((_QPW,), jnp.int32),
        pltpu.VMEM((_NB, _W, _D), jnp.float32),
        pltpu.VMEM((_W, 16), jnp.float32),
        pltpu.VMEM((_RPT, _D), jnp.float32),
        pltpu.VMEM((_RPT, 16), jnp.float32),
        pltpu.VMEM_SHARED((_NQ, _D), jnp.float32),
        pltpu.VMEM_SHARED((_NQ, _D), jnp.float32),
        pltpu.VMEM_SHARED((_NQ, 16), jnp.float32),
    ] + [pltpu.SemaphoreType.DMA] * _NB

    @functools.partial(pl.kernel, out_type=out_type, mesh=mesh,
                       scratch_types=scratch,
                       compiler_params=pltpu.CompilerParams(
                           use_tc_tiling_on_sc=False))
    def k(gx_h, bx_h, gc_h, qc_h, ga_h, t_h, qsum_h, csum_h, cnt_h, ae_h,
          gxi, bxi, gci, qci, gai, ring, ones, z64, z16, qacc, cacc, ctacc,
          *gsems):
        cid = lax.axis_index("c")
        sid = lax.axis_index("s")
        w = cid * 16 + sid
        row0 = sid * _RPT

        zero16 = jnp.zeros((16,), jnp.float32)
        one16 = jnp.full((16,), 1.0, jnp.float32)

        @pl.loop(0, _RPT)
        def _(r):
            for k4 in range(_D // 16):
                z64[r, pl.ds(k4 * 16, 16)] = zero16
            z16[r, :] = zero16

        @pl.loop(0, _W)
        def _(r):
            ones[r, :] = one16

        pltpu.sync_copy(z64, qacc.at[pl.ds(row0, _RPT)])
        pltpu.sync_copy(z64, cacc.at[pl.ds(row0, _RPT)])
        pltpu.sync_copy(z16, ctacc.at[pl.ds(row0, _RPT)])

        pltpu.sync_copy(gx_h.at[w], gxi)
        pltpu.sync_copy(bx_h.at[w], bxi)
        pltpu.sync_copy(gc_h.at[w], gci)
        pltpu.sync_copy(qc_h.at[w], qci)
        pltpu.sync_copy(ga_h.at[pl.ds(w * _QPW, _QPW)], gai)
        plsc.subcore_barrier()

        def start_g(idx_row, b):
            pltpu.async_copy(t_h.at[idx_row], ring.at[b], gsems[b])

        def wait_g(idx_row, b):
            pltpu.make_async_copy(t_h.at[idx_row], ring.at[b],
                                  gsems[b]).wait()

        def node_scatter(j, b):
            pltpu.sync_copy(ring.at[b], qacc.at[bxi.at[j]], add=True)
            pltpu.sync_copy(ones, ctacc.at[bxi.at[j]], add=True)

        for b in range(_NB):
            start_g(gxi.at[b], b)

        @pl.loop(0, _NODE_CH // _NB - 1)
        def _(g):
            for b in range(_NB):
                j = g * _NB + b
                wait_g(gxi.at[j], b)
                node_scatter(j, b)
                start_g(gxi.at[j + _NB], b)

        for b in range(_NB):
            j = _NODE_CH - _NB + b
            wait_g(gxi.at[j], b)
            node_scatter(j, b)
            start_g(gci.at[b], b)

        @pl.loop(0, _CORR_CH // _NB - 1)
        def _(g):
            for b in range(_NB):
                j = g * _NB + b
                wait_g(gci.at[j], b)
                pltpu.sync_copy(ring.at[b], cacc.at[qci.at[j]], add=True)
                start_g(gci.at[j + _NB], b)

        for b in range(_NB):
            j = _CORR_CH - _NB + b
            wait_g(gci.at[j], b)
            pltpu.sync_copy(ring.at[b], cacc.at[qci.at[j]], add=True)

        start_g(gai, 0)
        wait_g(gai, 0)
        pltpu.sync_copy(ring.at[0], ae_h.at[pl.ds(w * _QPW, _QPW)])

        plsc.subcore_barrier()
        pltpu.sync_copy(qacc.at[pl.ds(row0, _RPT)],
                        qsum_h.at[cid, pl.ds(row0, _RPT)])
        pltpu.sync_copy(cacc.at[pl.ds(row0, _RPT)],
                        csum_h.at[cid, pl.ds(row0, _RPT)])
        pltpu.sync_copy(ctacc.at[pl.ds(row0, _RPT)],
                        cnt_h.at[cid, pl.ds(row0, _RPT)])

    return k(gx, bx, gc, qc, ga, packed64)


def _tc_finish(qsum, csum, cnt, ansemb):
    def body(qs_ref, cs_ref, cn_ref, ae_ref, loss_ref, gold_ref, corr_ref):
        qs = qs_ref[0] + qs_ref[1]
        cs = cs_ref[0] + cs_ref[1]
        cn = cn_ref[0] + cn_ref[1]
        count = jnp.sum(cn, axis=1) * (1.0 / 16.0)
        query = qs / jnp.maximum(count, 1.0)[:, None]
        gold = jnp.sum(query * ae_ref[...], axis=1)
        corr = jnp.sum(query * cs, axis=1) * (1.0 / _NNEG)
        loss_ref[...] = jnp.mean(
            jnp.maximum(1.0 + corr - gold, 0.0)).reshape(1, 1)
        gold_ref[...] = gold
        corr_ref[...] = corr

    return pl.pallas_call(
        body,
        out_shape=[
            jax.ShapeDtypeStruct((1, 1), jnp.float32),
            jax.ShapeDtypeStruct((_NQ,), jnp.float32),
            jax.ShapeDtypeStruct((_NQ,), jnp.float32),
        ],
    )(qsum, csum, cnt, ansemb)


def _packed_row(e):
    return jnp.where(e < _H, 2 * e, 2 * (e - _H) + 1)


def kernel(x, batch, answers, corrupted, table):
    xf = x.astype(jnp.int32).reshape(_N_NODES)
    bf = batch.astype(jnp.int32)
    cf = corrupted.astype(jnp.int32).reshape(_NQ * _NNEG)
    af = answers.astype(jnp.int32)
    qid = jnp.arange(_NQ * _NNEG, dtype=jnp.int32) // _NNEG

    gx = _packed_row(xf).reshape(_NW, _NODE_CH, _W)
    bx = bf.reshape(_NW, _NODE_CH, _W)
    gc = _packed_row(cf).reshape(_NW, _CORR_CH, _W)
    qc = qid.reshape(_NW, _CORR_CH, _W)
    ga = _packed_row(af)

    packed = _tc_repack(table.astype(jnp.float32).T)
    packed64 = packed.reshape(2 * _H, _D)
    qsum, csum, cnt, ae = _sc_gather_accumulate(gx, bx, gc, qc, ga, packed64)
    loss, gold, corr = _tc_finish(qsum, csum, cnt, ae)
    return (loss.reshape(()), gold, corr)


# speedup vs baseline: 2.3963x; 1.0958x over previous
import functools

import jax
import jax.numpy as jnp
from jax import lax
from jax.experimental import pallas as pl
from jax.experimental.pallas import tpu as pltpu
from jax.experimental.pallas import tpu_sc as plsc

_D = 64
_N_NODES = 81920
_NQ = 4096
_NNEG = 50
_NW = 32
_W = 128

_L = 8192
_TGRID = 62
_H = _TGRID * _L
_LAST_B = 1000001 // _L

_NODE_CH = _N_NODES // _NW // _W
_CORR_CH = _NQ * _NNEG // _NW // _W
_NB = 5
_QPW = _NQ // _NW
_RPT = _NQ // 16


def _tc_repack(tabT):
    def body(a_ref, b_ref, out_ref):
        out_ref[:, 0:_D] = a_ref[...].T
        out_ref[:, _D:128] = b_ref[...].T

    return pl.pallas_call(
        body,
        grid=(_TGRID,),
        in_specs=[pl.BlockSpec((_D, _L), lambda i: (0, i)),
                  pl.BlockSpec((_D, _L),
                               lambda i: (0, jnp.minimum(i + _TGRID,
                                                         _LAST_B)))],
        out_specs=pl.BlockSpec((_L, 128), lambda i: (i, 0)),
        out_shape=jax.ShapeDtypeStruct((_H, 128), jnp.float32),
    )(tabT, tabT)


def _sc_gather_accumulate(gx, bx, gc, qc, ga, packed64):
    mesh = plsc.VectorSubcoreMesh(core_axis_name="c", subcore_axis_name="s")
    out_type = [
        jax.ShapeDtypeStruct((2, _NQ, _D), jnp.float32),
        jax.ShapeDtypeStruct((2, _NQ, _D), jnp.float32),
        jax.ShapeDtypeStruct((2, _NQ, 16), jnp.float32),
        jax.ShapeDtypeStruct((_NQ, _D), jnp.float32),
    ]
    scratch = [
        pltpu.VMEM((_NODE_CH, _W), jnp.int32),
        pltpu.VMEM((_NODE_CH, _W), jnp.int32),
        pltpu.VMEM((_CORR_CH, _W), jnp.int32),
        pltpu.VMEM((_CORR_CH, _W), jnp.int32),
        pltpu.VMEM((_QPW,), jnp.int32),
        pltpu.VMEM((_NB, _W, _D), jnp.float32),
        pltpu.VMEM((_W, 16), jnp.float32),
        pltpu.VMEM((_RPT, _D), jnp.float32),
        pltpu.VMEM((_RPT, 16), jnp.float32),
        pltpu.VMEM_SHARED((_NQ, _D), jnp.float32),
        pltpu.VMEM_SHARED((_NQ, _D), jnp.float32),
        pltpu.VMEM_SHARED((_NQ, 16), jnp.float32),
    ] + [pltpu.SemaphoreType.DMA] * _NB

    @functools.partial(pl.kernel, out_type=out_type, mesh=mesh,
                       scratch_types=scratch,
                       compiler_params=pltpu.CompilerParams(
                           use_tc_tiling_on_sc=False))
    def k(gx_h, bx_h, gc_h, qc_h, ga_h, t_h, qsum_h, csum_h, cnt_h, ae_h,
          gxi, bxi, gci, qci, gai, ring, ones, z64, z16, qacc, cacc, ctacc,
          *gsems):
        cid = lax.axis_index("c")
        sid = lax.axis_index("s")
        w = cid * 16 + sid
        row0 = sid * _RPT

        zero16 = jnp.zeros((16,), jnp.float32)
        one16 = jnp.full((16,), 1.0, jnp.float32)

        @pl.loop(0, _RPT)
        def _(r):
            for k4 in range(_D // 16):
                z64[r, pl.ds(k4 * 16, 16)] = zero16
            z16[r, :] = zero16

        @pl.loop(0, _W)
        def _(r):
            ones[r, :] = one16

        pltpu.sync_copy(z64, qacc.at[pl.ds(row0, _RPT)])
        pltpu.sync_copy(z64, cacc.at[pl.ds(row0, _RPT)])
        pltpu.sync_copy(z16, ctacc.at[pl.ds(row0, _RPT)])

        pltpu.sync_copy(gx_h.at[w], gxi)
        pltpu.sync_copy(bx_h.at[w], bxi)
        pltpu.sync_copy(gc_h.at[w], gci)
        pltpu.sync_copy(qc_h.at[w], qci)
        pltpu.sync_copy(ga_h.at[pl.ds(w * _QPW, _QPW)], gai)
        plsc.subcore_barrier()

        def start_g(idx_row, b):
            pltpu.async_copy(t_h.at[idx_row], ring.at[b], gsems[b])

        def wait_g(idx_row, b):
            pltpu.make_async_copy(t_h.at[idx_row], ring.at[b],
                                  gsems[b]).wait()

        def node_scatter(j, b):
            pltpu.sync_copy(ring.at[b], qacc.at[bxi.at[j]], add=True)
            pltpu.sync_copy(ones, ctacc.at[bxi.at[j]], add=True)

        for b in range(_NB):
            start_g(gxi.at[b], b)

        @pl.loop(0, _NODE_CH // _NB - 1)
        def _(g):
            for b in range(_NB):
                j = g * _NB + b
                wait_g(gxi.at[j], b)
                node_scatter(j, b)
                start_g(gxi.at[j + _NB], b)

        for b in range(_NB):
            j = _NODE_CH - _NB + b
            wait_g(gxi.at[j], b)
            node_scatter(j, b)
            start_g(gci.at[b], b)

        @pl.loop(0, _CORR_CH // _NB - 1)
        def _(g):
            for b in range(_NB):
                j = g * _NB + b
                wait_g(gci.at[j], b)
                pltpu.sync_copy(ring.at[b], cacc.at[qci.at[j]], add=True)
                start_g(gci.at[j + _NB], b)

        for b in range(_NB):
            j = _CORR_CH - _NB + b
            wait_g(gci.at[j], b)
            pltpu.sync_copy(ring.at[b], cacc.at[qci.at[j]], add=True)

        start_g(gai, 0)
        wait_g(gai, 0)
        pltpu.sync_copy(ring.at[0], ae_h.at[pl.ds(w * _QPW, _QPW)])

        plsc.subcore_barrier()
        pltpu.sync_copy(qacc.at[pl.ds(row0, _RPT)],
                        qsum_h.at[cid, pl.ds(row0, _RPT)])
        pltpu.sync_copy(cacc.at[pl.ds(row0, _RPT)],
                        csum_h.at[cid, pl.ds(row0, _RPT)])
        pltpu.sync_copy(ctacc.at[pl.ds(row0, _RPT)],
                        cnt_h.at[cid, pl.ds(row0, _RPT)])

    return k(gx, bx, gc, qc, ga, packed64)


def _tc_finish(qsum, csum, cnt, ansemb):
    def body(qs_ref, cs_ref, cn_ref, ae_ref, loss_ref, gold_ref, corr_ref):
        qs = qs_ref[0] + qs_ref[1]
        cs = cs_ref[0] + cs_ref[1]
        cn = cn_ref[0] + cn_ref[1]
        count = jnp.sum(cn, axis=1) * (1.0 / 16.0)
        query = qs / jnp.maximum(count, 1.0)[:, None]
        gold = jnp.sum(query * ae_ref[...], axis=1)
        corr = jnp.sum(query * cs, axis=1) * (1.0 / _NNEG)
        loss_ref[...] = jnp.mean(
            jnp.maximum(1.0 + corr - gold, 0.0)).reshape(1, 1)
        gold_ref[...] = gold
        corr_ref[...] = corr

    return pl.pallas_call(
        body,
        out_shape=[
            jax.ShapeDtypeStruct((1, 1), jnp.float32),
            jax.ShapeDtypeStruct((_NQ,), jnp.float32),
            jax.ShapeDtypeStruct((_NQ,), jnp.float32),
        ],
    )(qsum, csum, cnt, ansemb)


def _packed_row(e):
    return jnp.where(e < _H, 2 * e, 2 * (e - _H) + 1)


def kernel(x, batch, answers, corrupted, table):
    xf = x.astype(jnp.int32).reshape(_N_NODES)
    bf = batch.astype(jnp.int32)
    cf = corrupted.astype(jnp.int32).reshape(_NQ * _NNEG)
    af = answers.astype(jnp.int32)
    qid = jnp.arange(_NQ * _NNEG, dtype=jnp.int32) // _NNEG

    gx = _packed_row(xf).reshape(_NW, _NODE_CH, _W)
    bx = bf.reshape(_NW, _NODE_CH, _W)
    gc = _packed_row(cf).reshape(_NW, _CORR_CH, _W)
    qc = qid.reshape(_NW, _CORR_CH, _W)
    ga = _packed_row(af)

    packed = _tc_repack(table.astype(jnp.float32).T)
    packed64 = packed.reshape(2 * _H, _D)
    qsum, csum, cnt, ae = _sc_gather_accumulate(gx, bx, gc, qc, ga, packed64)
    loss, gold, corr = _tc_finish(qsum, csum, cnt, ae)
    return (loss.reshape(()), gold, corr)


# speedup vs baseline: 2.5079x; 1.0466x over previous
import functools

import jax
import jax.numpy as jnp
from jax import lax
from jax.experimental import pallas as pl
from jax.experimental.pallas import tpu as pltpu
from jax.experimental.pallas import tpu_sc as plsc

_D = 64
_N_NODES = 81920
_NQ = 4096
_NNEG = 50
_NW = 32
_W = 128

_L = 16384
_TGRID = 31
_H = _TGRID * _L
_LAST_B = 1000001 // _L

_NODE_CH = _N_NODES // _NW // _W
_CORR_CH = _NQ * _NNEG // _NW // _W
_NB = 5
_QPW = _NQ // _NW
_RPT = _NQ // 16


def _tc_repack(tabT):
    def body(a_ref, b_ref, out_ref):
        out_ref[:, 0:_D] = a_ref[...].T
        out_ref[:, _D:128] = b_ref[...].T

    return pl.pallas_call(
        body,
        grid=(_TGRID,),
        in_specs=[pl.BlockSpec((_D, _L), lambda i: (0, i)),
                  pl.BlockSpec((_D, _L),
                               lambda i: (0, jnp.minimum(i + _TGRID,
                                                         _LAST_B)))],
        out_specs=pl.BlockSpec((_L, 128), lambda i: (i, 0)),
        out_shape=jax.ShapeDtypeStruct((_H, 128), jnp.float32),
    )(tabT, tabT)


def _sc_gather_accumulate(gx, bx, gc, qc, ga, packed64):
    mesh = plsc.VectorSubcoreMesh(core_axis_name="c", subcore_axis_name="s")
    out_type = [
        jax.ShapeDtypeStruct((2, _NQ, _D), jnp.float32),
        jax.ShapeDtypeStruct((2, _NQ, _D), jnp.float32),
        jax.ShapeDtypeStruct((2, _NQ, 16), jnp.float32),
        jax.ShapeDtypeStruct((_NQ, _D), jnp.float32),
    ]
    scratch = [
        pltpu.VMEM((_NODE_CH, _W), jnp.int32),
        pltpu.VMEM((_NODE_CH, _W), jnp.int32),
        pltpu.VMEM((_CORR_CH, _W), jnp.int32),
        pltpu.VMEM((_CORR_CH, _W), jnp.int32),
        pltpu.VMEM((_QPW,), jnp.int32),
        pltpu.VMEM((_NB, _W, _D), jnp.float32),
        pltpu.VMEM((_W, 16), jnp.float32),
        pltpu.VMEM((_RPT, _D), jnp.float32),
        pltpu.VMEM((_RPT, 16), jnp.float32),
        pltpu.VMEM_SHARED((_NQ, _D), jnp.float32),
        pltpu.VMEM_SHARED((_NQ, _D), jnp.float32),
        pltpu.VMEM_SHARED((_NQ, 16), jnp.float32),
    ] + [pltpu.SemaphoreType.DMA] * _NB

    @functools.partial(pl.kernel, out_type=out_type, mesh=mesh,
                       scratch_types=scratch,
                       compiler_params=pltpu.CompilerParams(
                           use_tc_tiling_on_sc=False))
    def k(gx_h, bx_h, gc_h, qc_h, ga_h, t_h, qsum_h, csum_h, cnt_h, ae_h,
          gxi, bxi, gci, qci, gai, ring, ones, z64, z16, qacc, cacc, ctacc,
          *gsems):
        cid = lax.axis_index("c")
        sid = lax.axis_index("s")
        w = cid * 16 + sid
        row0 = sid * _RPT

        zero16 = jnp.zeros((16,), jnp.float32)
        one16 = jnp.full((16,), 1.0, jnp.float32)

        @pl.loop(0, _RPT)
        def _(r):
            for k4 in range(_D // 16):
                z64[r, pl.ds(k4 * 16, 16)] = zero16
            z16[r, :] = zero16

        @pl.loop(0, _W)
        def _(r):
            ones[r, :] = one16

        pltpu.sync_copy(z64, qacc.at[pl.ds(row0, _RPT)])
        pltpu.sync_copy(z64, cacc.at[pl.ds(row0, _RPT)])
        pltpu.sync_copy(z16, ctacc.at[pl.ds(row0, _RPT)])

        pltpu.sync_copy(gx_h.at[w], gxi)
        pltpu.sync_copy(bx_h.at[w], bxi)
        pltpu.sync_copy(gc_h.at[w], gci)
        pltpu.sync_copy(qc_h.at[w], qci)
        pltpu.sync_copy(ga_h.at[pl.ds(w * _QPW, _QPW)], gai)
        plsc.subcore_barrier()

        def start_g(idx_row, b):
            pltpu.async_copy(t_h.at[idx_row], ring.at[b], gsems[b])

        def wait_g(idx_row, b):
            pltpu.make_async_copy(t_h.at[idx_row], ring.at[b],
                                  gsems[b]).wait()

        def node_scatter(j, b):
            pltpu.sync_copy(ring.at[b], qacc.at[bxi.at[j]], add=True)
            pltpu.sync_copy(ones, ctacc.at[bxi.at[j]], add=True)

        for b in range(_NB):
            start_g(gxi.at[b], b)

        @pl.loop(0, _NODE_CH // _NB - 1)
        def _(g):
            for b in range(_NB):
                j = g * _NB + b
                wait_g(gxi.at[j], b)
                node_scatter(j, b)
                start_g(gxi.at[j + _NB], b)

        for b in range(_NB):
            j = _NODE_CH - _NB + b
            wait_g(gxi.at[j], b)
            node_scatter(j, b)
            start_g(gci.at[b], b)

        @pl.loop(0, _CORR_CH // _NB - 1)
        def _(g):
            for b in range(_NB):
                j = g * _NB + b
                wait_g(gci.at[j], b)
                pltpu.sync_copy(ring.at[b], cacc.at[qci.at[j]], add=True)
                start_g(gci.at[j + _NB], b)

        for b in range(_NB):
            j = _CORR_CH - _NB + b
            wait_g(gci.at[j], b)
            pltpu.sync_copy(ring.at[b], cacc.at[qci.at[j]], add=True)

        start_g(gai, 0)
        wait_g(gai, 0)
        pltpu.sync_copy(ring.at[0], ae_h.at[pl.ds(w * _QPW, _QPW)])

        plsc.subcore_barrier()
        pltpu.sync_copy(qacc.at[pl.ds(row0, _RPT)],
                        qsum_h.at[cid, pl.ds(row0, _RPT)])
        pltpu.sync_copy(cacc.at[pl.ds(row0, _RPT)],
                        csum_h.at[cid, pl.ds(row0, _RPT)])
        pltpu.sync_copy(ctacc.at[pl.ds(row0, _RPT)],
                        cnt_h.at[cid, pl.ds(row0, _RPT)])

    return k(gx, bx, gc, qc, ga, packed64)


def _tc_finish(qsum, csum, cnt, ansemb):
    def body(qs_ref, cs_ref, cn_ref, ae_ref, loss_ref, gold_ref, corr_ref):
        qs = qs_ref[0] + qs_ref[1]
        cs = cs_ref[0] + cs_ref[1]
        cn = cn_ref[0] + cn_ref[1]
        count = jnp.sum(cn, axis=1) * (1.0 / 16.0)
        query = qs / jnp.maximum(count, 1.0)[:, None]
        gold = jnp.sum(query * ae_ref[...], axis=1)
        corr = jnp.sum(query * cs, axis=1) * (1.0 / _NNEG)
        loss_ref[...] = jnp.mean(
            jnp.maximum(1.0 + corr - gold, 0.0)).reshape(1, 1)
        gold_ref[...] = gold
        corr_ref[...] = corr

    return pl.pallas_call(
        body,
        out_shape=[
            jax.ShapeDtypeStruct((1, 1), jnp.float32),
            jax.ShapeDtypeStruct((_NQ,), jnp.float32),
            jax.ShapeDtypeStruct((_NQ,), jnp.float32),
        ],
    )(qsum, csum, cnt, ansemb)


def _packed_row(e):
    return jnp.where(e < _H, 2 * e, 2 * (e - _H) + 1)


def kernel(x, batch, answers, corrupted, table):
    xf = x.astype(jnp.int32).reshape(_N_NODES)
    bf = batch.astype(jnp.int32)
    cf = corrupted.astype(jnp.int32).reshape(_NQ * _NNEG)
    af = answers.astype(jnp.int32)
    qid = jnp.arange(_NQ * _NNEG, dtype=jnp.int32) // _NNEG

    gx = _packed_row(xf).reshape(_NW, _NODE_CH, _W)
    bx = bf.reshape(_NW, _NODE_CH, _W)
    gc = _packed_row(cf).reshape(_NW, _CORR_CH, _W)
    qc = qid.reshape(_NW, _CORR_CH, _W)
    ga = _packed_row(af)

    packed = _tc_repack(table.astype(jnp.float32).T)
    packed64 = packed.reshape(2 * _H, _D)
    qsum, csum, cnt, ae = _sc_gather_accumulate(gx, bx, gc, qc, ga, packed64)
    loss, gold, corr = _tc_finish(qsum, csum, cnt, ae)
    return (loss.reshape(()), gold, corr)
